# baseline (speedup 1.0000x reference)
import functools

import jax
import jax.numpy as jnp
from jax.experimental import pallas as pl
from jax.experimental.pallas import tpu as pltpu

EPS = 1e-5
NUM_CLASSES = 10
H1 = W1 = 28
C1, C2 = 16, 32
H2 = W2 = 14
PR = 7
LX = 128
K1 = 2 * LX
CH1 = 256
N1 = 8 * CH1
CH2 = 256
K2 = 5 * CH2
CQ2 = 256
N2 = 2 * CQ2
FC_PAD = 128
FC_K = PR * CQ2


def _conv1_kernel(xl_ref, s1_ref, mm_ref, g_ref, rs_ref, *, bsz):
    t = pl.program_id(0)
    xl = xl_ref[...]
    lhs = jnp.concatenate([xl[0:PR], xl[1:PR + 1]],
                          axis=2).reshape(PR * bsz, K1)
    p = jnp.dot(lhs, s1_ref[...], preferred_element_type=jnp.float32)
    gram = jax.lax.dot_general(lhs, lhs, (((0,), (0,)), ((), ())),
                               preferred_element_type=jnp.float32)
    rsum = jnp.sum(lhs.astype(jnp.float32), axis=0, keepdims=True)

    @pl.when(t == 0)
    def _init():
        g_ref[...] = jnp.zeros_like(g_ref)
        rs_ref[...] = jnp.zeros_like(rs_ref)

    g_ref[...] += gram
    rs_ref[...] += rsum

    for par in range(2):
        c0 = p[:, (4 * par + 0) * CH1:(4 * par + 1) * CH1]
        c1 = p[:, (4 * par + 1) * CH1:(4 * par + 2) * CH1]
        c2 = p[:, (4 * par + 2) * CH1:(4 * par + 3) * CH1]
        c3 = p[:, (4 * par + 3) * CH1:(4 * par + 4) * CH1]
        pmax = jnp.maximum(jnp.maximum(c0, c1), jnp.maximum(c2, c3))
        pmin = jnp.minimum(jnp.minimum(c0, c1), jnp.minimum(c2, c3))
        mm_ref[2 * par] = pmax.reshape(PR, bsz, CH1)
        mm_ref[2 * par + 1] = pmin.reshape(PR, bsz, CH1)


def _conv2_kernel(mm_ref, scsh1_ref, s2_ref, mm2_ref, st_ref, *, bsz):
    t = pl.program_id(0)
    scp = scsh1_ref[0:1, :].reshape(1, 1, CH2)
    scn = scsh1_ref[1:2, :].reshape(1, 1, CH2)
    sh = scsh1_ref[2:3, :].reshape(1, 1, CH2)
    acts = []
    for par in range(2):
        pmax = mm_ref[2 * par]
        pmin = mm_ref[2 * par + 1]
        act = jnp.maximum(scp * pmax + scn * pmin + sh, 0.0)
        acts.append(act.astype(jnp.bfloat16))
    zrow = jnp.zeros((1, bsz, 2 * CH2), jnp.bfloat16)
    eo = jnp.concatenate(
        [zrow, jnp.concatenate(acts, axis=2), zrow], axis=0)
    s2 = s2_ref[...]
    lhs0 = jnp.concatenate(
        [eo[0:PR], eo[1:PR + 1], eo[2:PR + 2, :, 0:CH2]],
        axis=2).reshape(PR * bsz, K2)
    lhs1 = jnp.concatenate(
        [eo[0:PR, :, CH2:], eo[1:PR + 1], eo[2:PR + 2]],
        axis=2).reshape(PR * bsz, K2)
    p0 = jnp.dot(lhs0, s2, preferred_element_type=jnp.float32)
    p1 = jnp.dot(lhs1, s2, preferred_element_type=jnp.float32)

    tot = (jnp.sum(p0, axis=0, keepdims=True)
           + jnp.sum(p1, axis=0, keepdims=True))
    ssq = (jnp.sum(p0 * p0, axis=0, keepdims=True)
           + jnp.sum(p1 * p1, axis=0, keepdims=True))

    @pl.when(t == 0)
    def _init():
        st_ref[...] = jnp.zeros_like(st_ref)

    st_ref[0:1, :] += tot
    st_ref[1:2, :] += ssq

    pmax2 = jnp.maximum(jnp.maximum(p0[:, :CQ2], p0[:, CQ2:]),
                        jnp.maximum(p1[:, :CQ2], p1[:, CQ2:]))
    pmin2 = jnp.minimum(jnp.minimum(p0[:, :CQ2], p0[:, CQ2:]),
                        jnp.minimum(p1[:, :CQ2], p1[:, CQ2:]))
    mm2_ref[0] = pmax2.reshape(PR, bsz, CQ2)
    mm2_ref[1] = pmin2.reshape(PR, bsz, CQ2)


def _apply2_kernel(mm2_ref, scsh2_ref, wfc_ref, bfc_ref, o_ref, *, bsz):
    scp = scsh2_ref[0:1, :].reshape(1, 1, CQ2)
    scn = scsh2_ref[1:2, :].reshape(1, 1, CQ2)
    sh = scsh2_ref[2:3, :].reshape(1, 1, CQ2)
    pmax = mm2_ref[0]
    pmin = mm2_ref[1]
    act = jnp.maximum(scp * pmax + scn * pmin + sh, 0.0)
    lhs = jnp.concatenate([act[h] for h in range(PR)], axis=1)
    o_ref[...] = (jnp.dot(lhs, wfc_ref[...],
                          preferred_element_type=jnp.float32)
                  + bfc_ref[...])


def _conv1_call(xl, s1w, n, nblk, bsz):
    return pl.pallas_call(
        functools.partial(_conv1_kernel, bsz=bsz),
        grid=(nblk,),
        in_specs=[
            pl.BlockSpec((8, bsz, LX), lambda t: (0, t, 0)),
            pl.BlockSpec((K1, N1), lambda t: (0, 0)),
        ],
        out_specs=[
            pl.BlockSpec((4, PR, bsz, CH1), lambda t: (0, 0, t, 0)),
            pl.BlockSpec((K1, K1), lambda t: (0, 0)),
            pl.BlockSpec((1, K1), lambda t: (0, 0)),
        ],
        out_shape=[
            jax.ShapeDtypeStruct((4, PR, n, CH1), jnp.float32),
            jax.ShapeDtypeStruct((K1, K1), jnp.float32),
            jax.ShapeDtypeStruct((1, K1), jnp.float32),
        ],
        compiler_params=pltpu.CompilerParams(
            dimension_semantics=("arbitrary",)),
    )(xl, s1w)


def _conv2_call(mm1, scsh1, s2w, n, nblk, bsz):
    return pl.pallas_call(
        functools.partial(_conv2_kernel, bsz=bsz),
        grid=(nblk,),
        in_specs=[
            pl.BlockSpec((4, PR, bsz, CH1), lambda t: (0, 0, t, 0)),
            pl.BlockSpec((3, CH2), lambda t: (0, 0)),
            pl.BlockSpec((K2, N2), lambda t: (0, 0)),
        ],
        out_specs=[
            pl.BlockSpec((2, PR, bsz, CQ2), lambda t: (0, 0, t, 0)),
            pl.BlockSpec((2, N2), lambda t: (0, 0)),
        ],
        out_shape=[
            jax.ShapeDtypeStruct((2, PR, n, CQ2), jnp.float32),
            jax.ShapeDtypeStruct((2, N2), jnp.float32),
        ],
        compiler_params=pltpu.CompilerParams(
            dimension_semantics=("arbitrary",)),
    )(mm1, scsh1, s2w)


def _apply2_call(mm2, scsh2, wfc, bfc, n, nblk, bsz):
    return pl.pallas_call(
        functools.partial(_apply2_kernel, bsz=bsz),
        grid=(nblk,),
        in_specs=[
            pl.BlockSpec((2, PR, bsz, CQ2), lambda t: (0, 0, t, 0)),
            pl.BlockSpec((3, CQ2), lambda t: (0, 0)),
            pl.BlockSpec((FC_K, FC_PAD), lambda t: (0, 0)),
            pl.BlockSpec((1, FC_PAD), lambda t: (0, 0)),
        ],
        out_specs=pl.BlockSpec((bsz, FC_PAD), lambda t: (t, 0)),
        out_shape=jax.ShapeDtypeStruct((n, FC_PAD), jnp.float32),
        compiler_params=pltpu.CompilerParams(
            dimension_semantics=("arbitrary",)),
    )(mm2, scsh2, wfc, bfc)


def _minmax_trip(scale, shift, reps, lanes):
    trip = jnp.concatenate(
        [jnp.maximum(scale, 0.0), jnp.minimum(scale, 0.0), shift], axis=0)
    tiled = jnp.tile(trip, (1, reps))
    pad = lanes - tiled.shape[1]
    return jnp.pad(tiled, ((0, 0), (0, pad)))


def kernel(x_nchw, s1, t1, e1, g1, be1, s2, t2, e2, g2, be2, w_fc_rhc,
           b_fc_pad, w1_hwio, b1, w2_hwio, b2, g1_raw, be1_raw, g2_raw,
           be2_raw, w_fc, b_fc):
    n = x_nchw.shape[0]
    bsz = 256
    while n % bsz:
        bsz //= 2
    nblk = n // bsz
    bsza = 2 * bsz if n % (2 * bsz) == 0 else bsz
    nblka = n // bsza
    f32 = jnp.float32

    x = x_nchw.reshape(n, H1, W1).astype(jnp.bfloat16)
    xp = jnp.pad(x, ((0, 0), (2, 2), (2, 2)))
    xl = xp.reshape(n, 8, LX).transpose(1, 0, 2)

    s1r = s1.transpose(0, 2, 1, 3).reshape(5, 32, 2, 18, C1)[:, :, :, 2:16, :]
    s1r = jnp.pad(s1r.reshape(5, 32, 2, 14 * C1),
                  ((0, 0), (0, 0), (0, 0), (0, CH1 - 14 * C1)))
    s1r = s1r.reshape(5 * 32, 2 * CH1)
    zrb = [jnp.zeros((32 * k, 2 * CH1), f32) for k in range(4)]
    s1wf = jnp.concatenate(
        [jnp.concatenate([zrb[rb], s1r, zrb[3 - rb]], axis=0)
         for rb in range(4)], axis=1)
    s1w = s1wf.astype(jnp.bfloat16)

    s2r = s2.transpose(0, 2, 1, 3)
    s2r = s2r.reshape(5, 18, C1, 2, 224)[:, 2:16]
    s2r = jnp.pad(s2r.reshape(5, 14 * C1, 2, 224),
                  ((0, 0), (0, CH2 - 14 * C1), (0, 0), (0, CQ2 - 224)))
    s2w = s2r.reshape(K2, N2).astype(jnp.bfloat16)

    wfc = jnp.pad(w_fc_rhc, ((0, 0), (0, CQ2 - 224), (0, 0)))
    wfc = wfc.reshape(FC_K, FC_PAD)

    mm1, gram, rsum = _conv1_call(xl, s1w, n, nblk, bsz)

    s_lanes = (rsum @ s1wf).reshape(8, CH1).sum(axis=0)
    q_lanes = (s1wf * (gram @ s1wf)).sum(axis=0).reshape(8, CH1).sum(axis=0)
    s_ch = s_lanes.reshape(16, C1).sum(axis=0).reshape(1, C1)
    q_ch = q_lanes.reshape(16, C1).sum(axis=0).reshape(1, C1)
    inv1 = 1.0 / (n * H1 * W1)
    mean1 = s_ch * inv1
    var1 = q_ch * inv1 - mean1 * mean1
    scale1 = g1 * jax.lax.rsqrt(var1 + EPS)
    shift1 = be1 - scale1 * mean1
    scsh1 = _minmax_trip(scale1, shift1, 16, CH2)

    mm2, st2 = _conv2_call(mm1, scsh1, s2w, n, nblk, bsz)

    S2 = st2[:, 0:CQ2] + st2[:, CQ2:]
    ch2 = S2.reshape(2, 8, C2).sum(axis=1)
    inv2 = 1.0 / (n * H2 * W2)
    mean2 = ch2[0:1] * inv2
    var2 = ch2[1:2] * inv2 - mean2 * mean2
    scale2 = g2 * jax.lax.rsqrt(var2 + EPS)
    shift2 = be2 - scale2 * mean2
    scsh2 = _minmax_trip(scale2, shift2, 8, CQ2)

    logits = _apply2_call(mm2, scsh2, wfc, b_fc_pad, n, nblka, bsza)
    return logits[:, :NUM_CLASSES]
